# Initial kernel scaffold; baseline (speedup 1.0000x reference)
import jax
import jax.numpy as jnp
from jax.experimental import pallas as pl


def kernel(pred, target):
    raise NotImplementedError("write your pallas kernel here")


# speedup vs baseline: 9.2371x; 9.2371x over previous
import functools

import jax
import jax.numpy as jnp
import numpy as np
from jax.experimental import pallas as pl
from jax.experimental.pallas import tpu as pltpu

_N = 128
_KBINS = 100
_LPIX = 3.0


def _host_binning():
    kf = 2.0 * np.pi * np.fft.fftfreq(_N, d=_LPIX)
    kx, ky, kz = np.meshgrid(kf, kf, kf, indexing="ij")
    kmag = np.sqrt(kx * kx + ky * ky + kz * kz).ravel()
    pos = kmag > 0
    edges = np.linspace(kmag[pos].min(), kmag.max(), _KBINS + 1)
    ids = np.clip(np.digitize(kmag, edges) - 1, 0, _KBINS - 1)
    ids = np.where(pos, ids, _KBINS)
    counts = np.bincount(ids[pos], minlength=_KBINS).astype(np.float32)
    return ids.astype(np.int32).reshape(_N, _N, _N), counts


_IDS3, _COUNTS = _host_binning()
_IDS_ZF = np.ascontiguousarray(
    np.transpose(_IDS3, (2, 0, 1)).reshape(_N, _N * _N))
_MASK = np.zeros((1, 128), np.float32)
_MASK[0, :_KBINS] = (_COUNTS > 0).astype(np.float32)


def _dft_mats():
    j = np.arange(_N)
    ang = -2.0 * np.pi / _N * np.outer(j, j)
    cr = np.cos(ang)
    ci = np.sin(ang)

    def split(m):
        hi = m.astype(np.float32).astype(jnp.bfloat16)
        lo = (m.astype(np.float32) - np.asarray(hi, np.float32)).astype(
            jnp.bfloat16)
        return np.asarray(hi), np.asarray(lo)

    return split(cr) + split(ci)


_CRH, _CRL, _CIH, _CIL = _dft_mats()


def _split(x):
    hi = x.astype(jnp.bfloat16)
    lo = (x - hi.astype(jnp.float32)).astype(jnp.bfloat16)
    return hi, lo


def _mm(xh, xl, mh, ml):
    d = functools.partial(jnp.dot, preferred_element_type=jnp.float32)
    return d(xh, mh) + (d(xh, ml) + d(xl, mh))


def _fft_zy_kernel(x_ref, crh, crl, cih, cil, or_ref, oi_ref):
    t = x_ref[0]
    p2 = t.reshape(1024, 128)
    ph, plo = _split(p2)
    zr = _mm(ph, plo, crh[...], crl[...])
    zi = _mm(ph, plo, cih[...], cil[...])
    zr3 = jnp.transpose(zr.reshape(8, 128, 128), (0, 2, 1))
    zi3 = jnp.transpose(zi.reshape(8, 128, 128), (0, 2, 1))
    qr = zr3.reshape(1024, 128)
    qi = zi3.reshape(1024, 128)
    qrh, qrl = _split(qr)
    qih, qil = _split(qi)
    wr = _mm(qrh, qrl, crh[...], crl[...]) - _mm(qih, qil, cih[...], cil[...])
    wi = _mm(qrh, qrl, cih[...], cil[...]) + _mm(qih, qil, crh[...], crl[...])
    or_ref[0] = wr.reshape(8, 128, 128)
    oi_ref[0] = wi.reshape(8, 128, 128)


def _fft_zy(x):
    mat_spec = pl.BlockSpec(memory_space=pltpu.VMEM)
    blk = pl.BlockSpec((1, 8, 128, 128), lambda b, g: (b, g, 0, 0))
    return pl.pallas_call(
        _fft_zy_kernel,
        grid=(8, 16),
        in_specs=[blk, mat_spec, mat_spec, mat_spec, mat_spec],
        out_specs=[blk, blk],
        out_shape=[jax.ShapeDtypeStruct((8, 128, 128, 128), jnp.float32)] * 2,
        compiler_params=pltpu.CompilerParams(
            dimension_semantics=("parallel", "arbitrary")),
    )(x, _CRH, _CRL, _CIH, _CIL)


def _binpass_kernel(pr, pi, tr, ti, ids_ref, crh, crl, cih, cil, out_ref,
                    sar, sai):
    sums = jnp.zeros((16, 128), jnp.float32)
    for zi in range(8):
        ids_row = ids_ref[zi:zi + 1, :]
        oh = (jax.lax.broadcasted_iota(jnp.int32, (128, 16384), 0)
              == ids_row).astype(jnp.bfloat16)

        def copy_p(c, carry):
            lane = pl.multiple_of(c * 128, 128)
            sar[:, pl.ds(lane, 128)] = pr[c, :, zi, :]
            sai[:, pl.ds(lane, 128)] = pi[c, :, zi, :]
            return carry

        def copy_t(c, carry):
            lane = pl.multiple_of(c * 128 + 1024, 128)
            sar[:, pl.ds(lane, 128)] = tr[c, :, zi, :]
            sai[:, pl.ds(lane, 128)] = ti[c, :, zi, :]
            return carry

        jax.lax.fori_loop(0, 8, copy_p, 0)
        jax.lax.fori_loop(0, 8, copy_t, 0)

        srh, srl = _split(sar[...])
        sih, sil = _split(sai[...])
        vr = (_mm2(crh[...], crl[...], srh, srl)
              - _mm2(cih[...], cil[...], sih, sil))
        vi = (_mm2(cih[...], cil[...], srh, srl)
              + _mm2(crh[...], crl[...], sih, sil))
        pk = vr * vr + vi * vi
        rows = [pk[:, c * 128:(c + 1) * 128].reshape(1, 16384)
                .astype(jnp.bfloat16) for c in range(16)]
        pk16 = jnp.concatenate(rows, axis=0)
        sums = sums + jax.lax.dot_general(
            pk16, oh, (((1,), (1,)), ((), ())),
            preferred_element_type=jnp.float32)
    out_ref[0] = sums


def _mm2(mh, ml, xh, xl):
    d = functools.partial(jnp.dot, preferred_element_type=jnp.float32)
    return d(mh, xh) + (d(ml, xh) + d(mh, xl))


def _binpass(wpr, wpi, wtr, wti):
    mat_spec = pl.BlockSpec(memory_space=pltpu.VMEM)
    blk = pl.BlockSpec((8, 128, 8, 128), lambda j: (0, 0, j, 0))
    ids_blk = pl.BlockSpec((8, 16384), lambda j: (j, 0))
    out_blk = pl.BlockSpec((1, 16, 128), lambda j: (j, 0, 0))
    return pl.pallas_call(
        _binpass_kernel,
        grid=(16,),
        in_specs=[blk, blk, blk, blk, ids_blk,
                  mat_spec, mat_spec, mat_spec, mat_spec],
        out_specs=out_blk,
        out_shape=jax.ShapeDtypeStruct((16, 16, 128), jnp.float32),
        scratch_shapes=[pltpu.VMEM((128, 2048), jnp.float32),
                        pltpu.VMEM((128, 2048), jnp.float32)],
        compiler_params=pltpu.CompilerParams(
            dimension_semantics=("arbitrary",),
            vmem_limit_bytes=100 * 1024 * 1024),
    )(wpr, wpi, wtr, wti, jnp.asarray(_IDS_ZF), _CRH, _CRL, _CIH, _CIL)


def _finalize_kernel(part_ref, mask_ref, out_ref):
    p = part_ref[...]
    sums = p[0]
    for k in range(1, 16):
        sums = sums + p[k]
    sp = sums[0:8]
    st = sums[8:16]
    inv_ln10 = np.float32(1.0 / np.log(10.0))
    d = jnp.abs(jnp.log(st) - jnp.log(sp)) * inv_ln10
    m = mask_ref[...]
    dm = jnp.where(m > 0.0, d, 0.0)
    num = jnp.sum(dm, keepdims=True).reshape(1, 1)
    den = 8.0 * jnp.sum(m, keepdims=True).reshape(1, 1)
    out_ref[...] = num / den


def _finalize(partials):
    return pl.pallas_call(
        _finalize_kernel,
        in_specs=[pl.BlockSpec(memory_space=pltpu.VMEM)] * 2,
        out_specs=pl.BlockSpec(memory_space=pltpu.VMEM),
        out_shape=jax.ShapeDtypeStruct((1, 1), jnp.float32),
    )(partials, jnp.asarray(_MASK))


@jax.jit
def kernel(pred, target):
    xp = pred.reshape(8, 128, 128, 128)
    xt = target.reshape(8, 128, 128, 128)
    wpr, wpi = _fft_zy(xp)
    wtr, wti = _fft_zy(xt)
    partials = _binpass(wpr, wpi, wtr, wti)
    return _finalize(partials)[0, 0]


# speedup vs baseline: 9.2481x; 1.0012x over previous
import functools

import jax
import jax.numpy as jnp
import numpy as np
from jax.experimental import pallas as pl
from jax.experimental.pallas import tpu as pltpu

_N = 128
_KBINS = 100
_LPIX = 3.0


def _host_binning():
    kf = 2.0 * np.pi * np.fft.fftfreq(_N, d=_LPIX)
    kx, ky, kz = np.meshgrid(kf, kf, kf, indexing="ij")
    kmag = np.sqrt(kx * kx + ky * ky + kz * kz).ravel()
    pos = kmag > 0
    edges = np.linspace(kmag[pos].min(), kmag.max(), _KBINS + 1)
    ids = np.clip(np.digitize(kmag, edges) - 1, 0, _KBINS - 1)
    ids = np.where(pos, ids, _KBINS)
    counts = np.bincount(ids[pos], minlength=_KBINS).astype(np.float32)
    return ids.astype(np.int32).reshape(_N, _N, _N), counts


_IDS3, _COUNTS = _host_binning()
_IDS_ZF = np.ascontiguousarray(
    np.transpose(_IDS3, (2, 0, 1)).reshape(_N, _N * _N))
_MASK = np.zeros((1, 128), np.float32)
_MASK[0, :_KBINS] = (_COUNTS > 0).astype(np.float32)


def _dft_mats():
    j = np.arange(_N)
    ang = -2.0 * np.pi / _N * np.outer(j, j)
    cr = np.cos(ang)
    ci = np.sin(ang)

    def split(m):
        hi = m.astype(np.float32).astype(jnp.bfloat16)
        lo = (m.astype(np.float32) - np.asarray(hi, np.float32)).astype(
            jnp.bfloat16)
        return np.asarray(hi), np.asarray(lo)

    return split(cr) + split(ci)


_CRH, _CRL, _CIH, _CIL = _dft_mats()


def _split(x):
    hi = x.astype(jnp.bfloat16)
    lo = (x - hi.astype(jnp.float32)).astype(jnp.bfloat16)
    return hi, lo


def _mm(xh, xl, mh, ml):
    d = functools.partial(jnp.dot, preferred_element_type=jnp.float32)
    return d(xh, mh) + (d(xh, ml) + d(xl, mh))


def _fft_zy_kernel(x_ref, crh, crl, cih, cil, or_ref, oi_ref):
    t = x_ref[0]
    p2 = t.reshape(1024, 128)
    ph, plo = _split(p2)
    zr = _mm(ph, plo, crh[...], crl[...])
    zi = _mm(ph, plo, cih[...], cil[...])
    zr3 = jnp.transpose(zr.reshape(8, 128, 128), (0, 2, 1))
    zi3 = jnp.transpose(zi.reshape(8, 128, 128), (0, 2, 1))
    qr = zr3.reshape(1024, 128)
    qi = zi3.reshape(1024, 128)
    qrh, qrl = _split(qr)
    qih, qil = _split(qi)
    wr = _mm(qrh, qrl, crh[...], crl[...]) - _mm(qih, qil, cih[...], cil[...])
    wi = _mm(qrh, qrl, cih[...], cil[...]) + _mm(qih, qil, crh[...], crl[...])
    or_ref[0] = wr.reshape(8, 128, 128)
    oi_ref[0] = wi.reshape(8, 128, 128)


def _fft_zy(x):
    mat_spec = pl.BlockSpec(memory_space=pltpu.VMEM)
    blk = pl.BlockSpec((1, 8, 128, 128), lambda b, g: (b, g, 0, 0))
    return pl.pallas_call(
        _fft_zy_kernel,
        grid=(8, 16),
        in_specs=[blk, mat_spec, mat_spec, mat_spec, mat_spec],
        out_specs=[blk, blk],
        out_shape=[jax.ShapeDtypeStruct((8, 128, 128, 128), jnp.float32)] * 2,
        compiler_params=pltpu.CompilerParams(
            dimension_semantics=("parallel", "arbitrary")),
    )(x, _CRH, _CRL, _CIH, _CIL)


def _binpass_kernel(pr, pi, tr, ti, ids_ref, crh, crl, cih, cil, out_ref,
                    sar, sai):
    sums = jnp.zeros((16, 128), jnp.float32)
    for zi in range(8):
        ids_row = ids_ref[zi:zi + 1, :]
        oh = (jax.lax.broadcasted_iota(jnp.int32, (128, 16384), 0)
              == ids_row).astype(jnp.bfloat16)

        def copy_p(c, carry):
            lane = pl.multiple_of(c * 128, 128)
            sar[:, pl.ds(lane, 128)] = pr[c, :, zi, :]
            sai[:, pl.ds(lane, 128)] = pi[c, :, zi, :]
            return carry

        def copy_t(c, carry):
            lane = pl.multiple_of(c * 128 + 1024, 128)
            sar[:, pl.ds(lane, 128)] = tr[c, :, zi, :]
            sai[:, pl.ds(lane, 128)] = ti[c, :, zi, :]
            return carry

        jax.lax.fori_loop(0, 8, copy_p, 0)
        jax.lax.fori_loop(0, 8, copy_t, 0)

        srh, srl = _split(sar[...])
        sih, sil = _split(sai[...])
        vr = (_mm3(crh[...], crl[...], srh, srl)
              - _mm3(cih[...], cil[...], sih, sil))
        vi = (_mm3(cih[...], cil[...], srh, srl)
              + _mm3(crh[...], crl[...], sih, sil))
        pk = vr * vr + vi * vi
        rows = [pk[:, c * 128:(c + 1) * 128].reshape(1, 16384)
                .astype(jnp.bfloat16) for c in range(16)]
        pk16 = jnp.concatenate(rows, axis=0)
        sums = sums + jax.lax.dot_general(
            pk16, oh, (((1,), (1,)), ((), ())),
            preferred_element_type=jnp.float32)
    out_ref[0] = sums


def _mm3(mh, ml, xh, xl):
    d = functools.partial(jnp.dot, preferred_element_type=jnp.float32)
    return d(mh, xh) + (d(ml, xh) + d(mh, xl))


def _binpass(wpr, wpi, wtr, wti):
    mat_spec = pl.BlockSpec(memory_space=pltpu.VMEM)
    blk = pl.BlockSpec((8, 128, 8, 128), lambda j: (0, 0, j, 0))
    ids_blk = pl.BlockSpec((8, 16384), lambda j: (j, 0))
    out_blk = pl.BlockSpec((1, 16, 128), lambda j: (j, 0, 0))
    return pl.pallas_call(
        _binpass_kernel,
        grid=(16,),
        in_specs=[blk, blk, blk, blk, ids_blk,
                  mat_spec, mat_spec, mat_spec, mat_spec],
        out_specs=out_blk,
        out_shape=jax.ShapeDtypeStruct((16, 16, 128), jnp.float32),
        scratch_shapes=[pltpu.VMEM((128, 2048), jnp.float32),
                        pltpu.VMEM((128, 2048), jnp.float32)],
        compiler_params=pltpu.CompilerParams(
            dimension_semantics=("parallel",),
            vmem_limit_bytes=100 * 1024 * 1024),
    )(wpr, wpi, wtr, wti, jnp.asarray(_IDS_ZF), _CRH, _CRL, _CIH, _CIL)


def _finalize_kernel(part_ref, mask_ref, out_ref):
    p = part_ref[...]
    sums = p[0]
    for k in range(1, 16):
        sums = sums + p[k]
    sp = sums[0:8]
    st = sums[8:16]
    inv_ln10 = np.float32(1.0 / np.log(10.0))
    d = jnp.abs(jnp.log(st) - jnp.log(sp)) * inv_ln10
    m = mask_ref[...]
    dm = jnp.where(m > 0.0, d, 0.0)
    num = jnp.sum(dm, keepdims=True).reshape(1, 1)
    den = 8.0 * jnp.sum(m, keepdims=True).reshape(1, 1)
    out_ref[...] = num / den


def _finalize(partials):
    return pl.pallas_call(
        _finalize_kernel,
        in_specs=[pl.BlockSpec(memory_space=pltpu.VMEM)] * 2,
        out_specs=pl.BlockSpec(memory_space=pltpu.VMEM),
        out_shape=jax.ShapeDtypeStruct((1, 1), jnp.float32),
    )(partials, jnp.asarray(_MASK))


@jax.jit
def kernel(pred, target):
    xp = pred.reshape(8, 128, 128, 128)
    xt = target.reshape(8, 128, 128, 128)
    wpr, wpi = _fft_zy(xp)
    wtr, wti = _fft_zy(xt)
    partials = _binpass(wpr, wpi, wtr, wti)
    return _finalize(partials)[0, 0]
